# baseline (speedup 1.0000x reference)
import math
import functools

import jax
import jax.numpy as jnp
from jax.experimental import pallas as pl
from jax.experimental.pallas import tpu as pltpu

H = 4
C = 32
HID = H * C
NEG_SLOPE = 0.2
POOL_RATIO = 0.5

ROW_BLK = 256


def _mm_bias_kernel(x_ref, w_ref, b_ref, o_ref):
    o_ref[...] = (
        jnp.dot(x_ref[...], w_ref[...], preferred_element_type=jnp.float32)
        + b_ref[...]
    )


def _mm_bias(x, w, b):
    n, d_in = x.shape
    d_out = w.shape[1]
    n_pad = ((n + ROW_BLK - 1) // ROW_BLK) * ROW_BLK
    xp = jnp.pad(x, ((0, n_pad - n), (0, 0)))
    out = pl.pallas_call(
        _mm_bias_kernel,
        grid=(n_pad // ROW_BLK,),
        in_specs=[
            pl.BlockSpec((ROW_BLK, d_in), lambda i: (i, 0)),
            pl.BlockSpec((d_in, d_out), lambda i: (0, 0)),
            pl.BlockSpec((1, d_out), lambda i: (0, 0)),
        ],
        out_specs=pl.BlockSpec((ROW_BLK, d_out), lambda i: (i, 0)),
        out_shape=jax.ShapeDtypeStruct((n_pad, d_out), jnp.float32),
    )(xp, w, b.reshape(1, d_out))
    return out[:n]


def _layer_norm(x, g, b):
    mu = jnp.mean(x, axis=-1, keepdims=True)
    var = jnp.mean((x - mu) ** 2, axis=-1, keepdims=True)
    return (x - mu) / jnp.sqrt(var + 1e-5) * g + b


def _gatv2(x, src, dst, emask, Wl, bl, Wr, br, att, bias, n):
    xl = _mm_bias(x, Wl, bl).reshape(-1, H, C)
    xr = _mm_bias(x, Wr, br).reshape(-1, H, C)
    m = xl[src] + xr[dst]
    e = jnp.where(m >= 0, m, NEG_SLOPE * m)
    logit = jnp.sum(e * att[None, :, :], axis=-1)
    logit = jnp.where(emask[:, None], logit, -1e30)
    mx = jax.ops.segment_max(logit, dst, num_segments=n)
    mx = jnp.where(jnp.isfinite(mx), mx, 0.0)
    ex = jnp.exp(logit - mx[dst]) * emask[:, None].astype(x.dtype)
    den = jax.ops.segment_sum(ex, dst, num_segments=n)
    alpha = ex / (den[dst] + 1e-16)
    out = jax.ops.segment_sum(xl[src] * alpha[:, :, None], dst, num_segments=n)
    return out.reshape(n, H * C) + bias


def _stack(x, src, dst, emask, Wl, bl, Wr, br, att, bias, lng, lnb, n):
    out = x
    for i in range(Wl.shape[0]):
        h = jax.nn.relu(_gatv2(out, src, dst, emask, Wl[i], bl[i], Wr[i], br[i], att[i], bias[i], n))
        h = _layer_norm(h, lng[i], lnb[i])
        out = h + out
    return out


def kernel(x, edge_index, batch, type_id, global_token, bb_Wl, bb_bl, bb_Wr, bb_br, bb_att, bb_bias, bb_lng, bb_lnb, top_Wl, top_bl, top_Wr, top_br, top_att, top_bias, top_lng, top_lnb, pool_w, type_emb, fam_W, fam_b, typ_W, typ_b):
    n0 = x.shape[0]
    xv = jnp.concatenate([x, global_token], axis=0)
    nodes = jnp.arange(n0, dtype=edge_index.dtype)
    tokv = jnp.full((n0,), n0, dtype=edge_index.dtype)
    src = jnp.concatenate([edge_index[0], nodes, tokv])
    dst = jnp.concatenate([edge_index[1], tokv, nodes])
    n_tot = n0 + 1
    full_mask = jnp.ones(src.shape[0], dtype=bool)
    h = _stack(xv, src, dst, full_mask, bb_Wl, bb_bl, bb_Wr, bb_br, bb_att, bb_bias, bb_lng, bb_lnb, n_tot)
    score = jnp.tanh(jnp.sum(h * pool_w[None, :], axis=-1) / jnp.linalg.norm(pool_w))
    k = int(math.ceil(POOL_RATIO * n_tot))
    top_s, perm = jax.lax.top_k(score, k)
    hp = h[perm] * top_s[:, None]
    keep = jnp.zeros(n_tot, dtype=bool).at[perm].set(True)
    new_id = jnp.zeros(n_tot, dtype=src.dtype).at[perm].set(jnp.arange(k, dtype=src.dtype))
    emask2 = keep[src] & keep[dst]
    src2 = jnp.where(emask2, new_id[src], 0)
    dst2 = jnp.where(emask2, new_id[dst], 0)
    h2 = _stack(hp, src2, dst2, emask2, top_Wl, top_bl, top_Wr, top_br, top_att, top_bias, top_lng, top_lnb, k)
    g = jnp.mean(h2, axis=0, keepdims=True)
    g = jnp.concatenate([g, type_emb[type_id]], axis=1)
    return (_mm_bias(g, fam_W, fam_b), _mm_bias(g, typ_W, typ_b))
